# baseline (speedup 1.0000x reference)
import functools

import jax
import jax.numpy as jnp
from jax import lax
from jax.experimental import pallas as pl
from jax.experimental.pallas import tpu as pltpu
from jax.experimental.pallas import tpu_sc as plsc

B = 16384
D = 512
L = 16
NC = 2
NS = 16
NW = NC * NS
BPW = B // NW
C = 64
G = C // L
NCHUNKS = BPW // C


def _sc_body(table, idx_a, idx_b, wts, x0_out, x1_out,
             idx_av, idx_bv, rows_a, rows_b, wts_v, x0_c, x1_c,
             sem_a, sem_b):
  wid = lax.axis_index("s") * NC + lax.axis_index("c")
  base0 = wid * BPW

  pltpu.sync_copy(wts, wts_v)

  iota = lax.iota(jnp.int32, L)
  zero = jnp.zeros((L,), jnp.float32)

  def chunk_body(c, carry):
    base = base0 + c * C
    pltpu.sync_copy(idx_a.at[pl.ds(base, C)], idx_av)
    pltpu.sync_copy(idx_b.at[pl.ds(base, C)], idx_bv)
    cpa = pltpu.async_copy(table.at[idx_av], rows_a, sem_a)
    cpb = pltpu.async_copy(table.at[idx_bv], rows_b, sem_b)
    cpa.wait()
    cpb.wait()

    def j_body(j, accs):
      js = jnp.full((L,), j, jnp.int32)
      w0 = plsc.load_gather(wts_v, [js])
      w1 = plsc.load_gather(wts_v, [js + D])
      w2 = plsc.load_gather(wts_v, [js + 2 * D])
      w3 = plsc.load_gather(wts_v, [js + 3 * D])
      out = []
      for g in range(G):
        ia = iota + (g * L)
        va = plsc.load_gather(rows_a, [ia, js])
        vb = plsc.load_gather(rows_b, [ia, js])
        out.append(accs[2 * g] + (va * w0 + vb * w2))
        out.append(accs[2 * g + 1] + (va * w1 + vb * w3))
      return tuple(out)

    accs = lax.fori_loop(0, D, j_body, tuple(zero for _ in range(2 * G)))
    for g in range(G):
      x0_c[pl.ds(g * L, L)] = accs[2 * g]
      x1_c[pl.ds(g * L, L)] = accs[2 * g + 1]

    pltpu.sync_copy(x0_c, x0_out.at[pl.ds(base, C)])
    pltpu.sync_copy(x1_c, x1_out.at[pl.ds(base, C)])
    return carry

  lax.fori_loop(0, NCHUNKS, chunk_body, 0)


@jax.jit
def _sc_logits(table, idx_a, idx_b, wts):
  mesh = plsc.VectorSubcoreMesh(core_axis_name="c", subcore_axis_name="s")
  kern = functools.partial(
      pl.kernel,
      mesh=mesh,
      compiler_params=pltpu.CompilerParams(needs_layout_passes=False),
      out_type=(
          jax.ShapeDtypeStruct((B,), jnp.float32),
          jax.ShapeDtypeStruct((B,), jnp.float32),
      ),
      scratch_types=[
          pltpu.VMEM((C,), jnp.int32),
          pltpu.VMEM((C,), jnp.int32),
          pltpu.VMEM((C, D), jnp.float32),
          pltpu.VMEM((C, D), jnp.float32),
          pltpu.VMEM((4 * D,), jnp.float32),
          pltpu.VMEM((C,), jnp.float32),
          pltpu.VMEM((C,), jnp.float32),
          pltpu.SemaphoreType.DMA,
          pltpu.SemaphoreType.DMA,
      ],
  )(_sc_body)
  return kern(table, idx_a, idx_b, wts)


def _tc_body(x0_ref, x1_ref, b3_ref, o0_ref, o1_ref):
  x0 = x0_ref[...] + b3_ref[0]
  x1 = x1_ref[...] + b3_ref[1]
  m = jnp.maximum(x0, x1)
  lse = m + jnp.log(jnp.exp(x0 - m) + jnp.exp(x1 - m))
  o0_ref[...] = x0 - lse
  o1_ref[...] = x1 - lse


@jax.jit
def _tc_logsoftmax(x0, x1, b3):
  r = B // 128
  o0, o1 = pl.pallas_call(
      _tc_body,
      out_shape=(
          jax.ShapeDtypeStruct((r, 128), jnp.float32),
          jax.ShapeDtypeStruct((r, 128), jnp.float32),
      ),
      in_specs=[
          pl.BlockSpec(memory_space=pltpu.VMEM),
          pl.BlockSpec(memory_space=pltpu.VMEM),
          pl.BlockSpec(memory_space=pltpu.SMEM),
      ],
      out_specs=(
          pl.BlockSpec(memory_space=pltpu.VMEM),
          pl.BlockSpec(memory_space=pltpu.VMEM),
      ),
  )(x0.reshape(r, 128), x1.reshape(r, 128), b3)
  return jnp.stack([o0.reshape(B), o1.reshape(B)], axis=-1)


def kernel(inputs, epoch, table, W3, b3):
  del epoch
  w0 = W3[0:512]
  w1 = W3[512:1024]
  w2 = W3[1024:1536]
  w3b = W3[1536:2048]
  wa = w0 + w1 + w2
  wb = w1 - w0 + w3b
  wts = jnp.concatenate([wa[:, 0], wa[:, 1], wb[:, 0], wb[:, 1]])
  idx_a = inputs[:, 0]
  idx_b = inputs[:, 1]
  x0, x1 = _sc_logits(table, idx_a, idx_b, wts)
  return _tc_logsoftmax(x0, x1, b3)
